# baseline (speedup 1.0000x reference)
import functools

import jax
import jax.numpy as jnp
from jax import lax
from jax.experimental import pallas as pl
from jax.experimental.pallas import tpu as pltpu
from jax.experimental.pallas import tpu_sc as plsc

_NC = 2
_NS = 16
_NW = _NC * _NS
_CH = 128


def _sc_mesh():
  return plsc.VectorSubcoreMesh(core_axis_name="c", subcore_axis_name="s")


def _sc_aggregate(hs, eidx, n_pad, nch):
  d = hs.shape[1]
  rpt = n_pad // _NS

  @functools.partial(
      pl.kernel,
      out_type=jax.ShapeDtypeStruct((_NC, n_pad, d), jnp.float32),
      mesh=_sc_mesh(),
      scratch_types=[
          pltpu.VMEM_SHARED((n_pad, d), jnp.float32),
          pltpu.VMEM((2, 1, _CH), jnp.int32),
          pltpu.VMEM((2, 1, _CH), jnp.int32),
          pltpu.VMEM((_CH, d), jnp.float32),
          pltpu.VMEM((_CH, d), jnp.float32),
          pltpu.SemaphoreType.DMA,
          pltpu.SemaphoreType.DMA,
      ],
  )
  def k(hs_hbm, eidx_hbm, out_hbm, agg_sh, idx_a, idx_b, buf_a, buf_b,
        sem_a, sem_b):
    c = lax.axis_index("c")
    s = lax.axis_index("s")
    w = c * _NS + s
    pltpu.sync_copy(hs_hbm.at[pl.ds(s * rpt, rpt)],
                    agg_sh.at[pl.ds(s * rpt, rpt)])
    plsc.subcore_barrier()

    pltpu.sync_copy(eidx_hbm.at[:, w, 0], idx_a)
    pltpu.async_copy(hs_hbm.at[idx_a.at[0, 0]], buf_a, sem_a)

    @pl.loop(0, nch, step=2)
    def _(j):
      pltpu.sync_copy(eidx_hbm.at[:, w, j + 1], idx_b)
      pltpu.async_copy(hs_hbm.at[idx_b.at[0, 0]], buf_b, sem_b)
      pltpu.make_async_copy(hs_hbm.at[idx_a.at[0, 0]], buf_a, sem_a).wait()
      pltpu.sync_copy(buf_a, agg_sh.at[idx_a.at[1, 0]], add=True)

      @pl.when(j + 2 < nch)
      def _():
        pltpu.sync_copy(eidx_hbm.at[:, w, j + 2], idx_a)
        pltpu.async_copy(hs_hbm.at[idx_a.at[0, 0]], buf_a, sem_a)

      pltpu.make_async_copy(hs_hbm.at[idx_b.at[0, 0]], buf_b, sem_b).wait()
      pltpu.sync_copy(buf_b, agg_sh.at[idx_b.at[1, 0]], add=True)

    plsc.subcore_barrier()
    pltpu.sync_copy(agg_sh.at[pl.ds(s * rpt, rpt)],
                    out_hbm.at[c, pl.ds(s * rpt, rpt)])

  return k(hs, eidx)


def _dinv_col(degp_a, degp_b, n):
  deg = degp_a + degp_b - 1.0
  return lax.rsqrt(jnp.maximum(deg, 1.0))[:n, 0:1]


def _tc_first(x, w1, degp, n_pad):
  n, d_in = x.shape
  d = w1.shape[1]

  def body(x_ref, w_ref, degp_ref, hs_ref):
    dinv = _dinv_col(degp_ref[0], degp_ref[1], n)
    h = jnp.dot(x_ref[...], w_ref[...], preferred_element_type=jnp.float32)
    hs_ref[0:n, :] = h * dinv
    hs_ref[n:n_pad, :] = jnp.zeros((n_pad - n, d), jnp.float32)

  return pl.pallas_call(
      body,
      out_shape=jax.ShapeDtypeStruct((n_pad, d), jnp.float32),
  )(x, w1, degp)


def _tc_mid(aggp, hs1, degp, b1, w2, n):
  n_pad, d = hs1.shape

  def body(aggp_ref, hs1_ref, degp_ref, b_ref, w_ref, hs2_ref):
    dinv_full = lax.rsqrt(
        jnp.maximum(degp_ref[0] + degp_ref[1] - 1.0, 1.0))[:, 0:1]
    agg = aggp_ref[0] + aggp_ref[1] - hs1_ref[...]
    h1 = jnp.maximum(dinv_full * agg + b_ref[...], 0.0)
    hs2 = jnp.dot(h1, w_ref[...],
                  preferred_element_type=jnp.float32) * dinv_full
    hs2_ref[0:n, :] = hs2[0:n, :]
    hs2_ref[n:n_pad, :] = jnp.zeros((n_pad - n, d), jnp.float32)

  return pl.pallas_call(
      body,
      out_shape=jax.ShapeDtypeStruct((n_pad, d), jnp.float32),
  )(aggp, hs1, degp, b1, w2)


def _tc_last(aggp, hs2, degp, b2, n):
  n_pad, d = hs2.shape

  def body(aggp_ref, hs2_ref, degp_ref, b_ref, out_ref):
    dinv = _dinv_col(degp_ref[0], degp_ref[1], n)
    agg = (aggp_ref[0] + aggp_ref[1] - hs2_ref[...])[0:n, :]
    v = dinv * agg + b_ref[...]
    shifted = v - jnp.max(v, axis=1, keepdims=True)
    out_ref[...] = shifted - jnp.log(
        jnp.sum(jnp.exp(shifted), axis=1, keepdims=True))

  return pl.pallas_call(
      body,
      out_shape=jax.ShapeDtypeStruct((n, d), jnp.float32),
  )(aggp, hs2, degp, b2)


@jax.jit
def kernel(x, edge_index, W1, b1, W2, b2):
  n = x.shape[0]
  e = edge_index.shape[1]

  nch = -(-e // (_NW * _CH))
  nch = ((nch + 7) // 8) * 8
  e_pad = _NW * _CH * nch
  n_pad = ((n + 1 + _NS * 8 - 1) // (_NS * 8)) * (_NS * 8)
  src = edge_index[0].astype(jnp.int32)
  dst = edge_index[1].astype(jnp.int32)
  if e_pad != e:
    fill = jnp.full((e_pad - e,), n, dtype=jnp.int32)
    src = jnp.concatenate([src, fill])
    dst = jnp.concatenate([dst, fill])
  eidx = jnp.stack([src, dst]).reshape(2, _NW, nch, 1, _CH)
  ones128 = jnp.ones((n_pad, 128), jnp.float32)

  degp = _sc_aggregate(ones128, eidx, n_pad, nch)
  hs1 = _tc_first(x, W1, degp, n_pad)
  aggp1 = _sc_aggregate(hs1, eidx, n_pad, nch)
  hs2 = _tc_mid(aggp1, hs1, degp, b1.reshape(1, -1), W2, n)
  aggp2 = _sc_aggregate(hs2, eidx, n_pad, nch)
  return _tc_last(aggp2, hs2, degp, b2.reshape(1, -1), n)


# speedup vs baseline: 1.3057x; 1.3057x over previous
import functools

import jax
import jax.numpy as jnp
from jax import lax
from jax.experimental import pallas as pl
from jax.experimental.pallas import tpu as pltpu
from jax.experimental.pallas import tpu_sc as plsc

_NC = 2
_NS = 16
_NW = _NC * _NS
_CH = 64


def _sc_mesh():
  return plsc.VectorSubcoreMesh(core_axis_name="c", subcore_axis_name="s")


def _sc_aggregate(hs, eidx, n_pad, nch):
  d = hs.shape[1]
  rpt = n_pad // _NS

  @functools.partial(
      pl.kernel,
      out_type=jax.ShapeDtypeStruct((_NC, n_pad, d), jnp.float32),
      mesh=_sc_mesh(),
      scratch_types=[
          pltpu.VMEM_SHARED((n_pad, d), jnp.float32),
          pltpu.VMEM((2, 1, _CH), jnp.int32),
          pltpu.VMEM((2, 1, _CH), jnp.int32),
          pltpu.VMEM((_CH, d), jnp.float32),
          pltpu.VMEM((_CH, d), jnp.float32),
          pltpu.SemaphoreType.DMA,
          pltpu.SemaphoreType.DMA,
      ],
  )
  def k(hs_hbm, eidx_hbm, out_hbm, agg_sh, idx_a, idx_b, buf_a, buf_b,
        sem_a, sem_b):
    c = lax.axis_index("c")
    s = lax.axis_index("s")
    w = c * _NS + s
    pltpu.sync_copy(hs_hbm.at[pl.ds(s * rpt, rpt)],
                    agg_sh.at[pl.ds(s * rpt, rpt)])
    plsc.subcore_barrier()

    pltpu.sync_copy(eidx_hbm.at[:, w, 0], idx_a)
    pltpu.async_copy(hs_hbm.at[idx_a.at[0, 0]], buf_a, sem_a)

    @pl.loop(0, nch, step=2)
    def _(j):
      pltpu.sync_copy(eidx_hbm.at[:, w, j + 1], idx_b)
      pltpu.async_copy(hs_hbm.at[idx_b.at[0, 0]], buf_b, sem_b)
      pltpu.make_async_copy(hs_hbm.at[idx_a.at[0, 0]], buf_a, sem_a).wait()
      pltpu.sync_copy(buf_a, agg_sh.at[idx_a.at[1, 0]], add=True)

      @pl.when(j + 2 < nch)
      def _():
        pltpu.sync_copy(eidx_hbm.at[:, w, j + 2], idx_a)
        pltpu.async_copy(hs_hbm.at[idx_a.at[0, 0]], buf_a, sem_a)

      pltpu.make_async_copy(hs_hbm.at[idx_b.at[0, 0]], buf_b, sem_b).wait()
      pltpu.sync_copy(buf_b, agg_sh.at[idx_b.at[1, 0]], add=True)

    plsc.subcore_barrier()
    pltpu.sync_copy(agg_sh.at[pl.ds(s * rpt, rpt)],
                    out_hbm.at[c, pl.ds(s * rpt, rpt)])

  return k(hs, eidx)


def _sc_degree(eidx, zeros_nd, ones_chd, n_pad, nch):
  dd = zeros_nd.shape[1]
  rpt = n_pad // _NS

  @functools.partial(
      pl.kernel,
      out_type=jax.ShapeDtypeStruct((_NC, n_pad, dd), jnp.float32),
      mesh=_sc_mesh(),
      scratch_types=[
          pltpu.VMEM_SHARED((n_pad, dd), jnp.float32),
          pltpu.VMEM((1, _CH), jnp.int32),
          pltpu.VMEM((_CH, dd), jnp.float32),
      ],
  )
  def k(eidx_hbm, zeros_hbm, ones_hbm, out_hbm, agg_sh, idx, ones_buf):
    c = lax.axis_index("c")
    s = lax.axis_index("s")
    w = c * _NS + s
    pltpu.sync_copy(zeros_hbm.at[pl.ds(s * rpt, rpt)],
                    agg_sh.at[pl.ds(s * rpt, rpt)])
    pltpu.sync_copy(ones_hbm, ones_buf)
    plsc.subcore_barrier()

    @pl.loop(0, nch)
    def _(j):
      pltpu.sync_copy(eidx_hbm.at[1, w, j], idx)
      pltpu.sync_copy(ones_buf, agg_sh.at[idx.at[0]], add=True)

    plsc.subcore_barrier()
    pltpu.sync_copy(agg_sh.at[pl.ds(s * rpt, rpt)],
                    out_hbm.at[c, pl.ds(s * rpt, rpt)])

  return k(eidx, zeros_nd, ones_chd)


def _dinv_col(degp_a, degp_b, n):
  deg = degp_a + degp_b + 1.0
  return lax.rsqrt(jnp.maximum(deg, 1.0))[:n, 0:1]


def _tc_first(x, w1, degp, n_pad):
  n, d_in = x.shape
  d = w1.shape[1]

  def body(x_ref, w_ref, degp_ref, hs_ref):
    dinv = _dinv_col(degp_ref[0], degp_ref[1], n)
    h = jnp.dot(x_ref[...], w_ref[...], preferred_element_type=jnp.float32)
    hs_ref[0:n, :] = h * dinv
    hs_ref[n:n_pad, :] = jnp.zeros((n_pad - n, d), jnp.float32)

  return pl.pallas_call(
      body,
      out_shape=jax.ShapeDtypeStruct((n_pad, d), jnp.float32),
  )(x, w1, degp)


def _tc_mid(aggp, hs1, degp, b1, w2, n):
  n_pad, d = hs1.shape

  def body(aggp_ref, hs1_ref, degp_ref, b_ref, w_ref, hs2_ref):
    dinv_full = lax.rsqrt(
        jnp.maximum(degp_ref[0] + degp_ref[1] + 1.0, 1.0))[:, 0:1]
    agg = aggp_ref[0] + aggp_ref[1] - hs1_ref[...]
    h1 = jnp.maximum(dinv_full * agg + b_ref[...], 0.0)
    hs2 = jnp.dot(h1, w_ref[...],
                  preferred_element_type=jnp.float32) * dinv_full
    hs2_ref[0:n, :] = hs2[0:n, :]
    hs2_ref[n:n_pad, :] = jnp.zeros((n_pad - n, d), jnp.float32)

  return pl.pallas_call(
      body,
      out_shape=jax.ShapeDtypeStruct((n_pad, d), jnp.float32),
  )(aggp, hs1, degp, b1, w2)


def _tc_last(aggp, hs2, degp, b2, n):
  n_pad, d = hs2.shape

  def body(aggp_ref, hs2_ref, degp_ref, b_ref, out_ref):
    dinv = _dinv_col(degp_ref[0], degp_ref[1], n)
    agg = (aggp_ref[0] + aggp_ref[1] - hs2_ref[...])[0:n, :]
    v = dinv * agg + b_ref[...]
    shifted = v - jnp.max(v, axis=1, keepdims=True)
    out_ref[...] = shifted - jnp.log(
        jnp.sum(jnp.exp(shifted), axis=1, keepdims=True))

  return pl.pallas_call(
      body,
      out_shape=jax.ShapeDtypeStruct((n, d), jnp.float32),
  )(aggp, hs2, degp, b2)


@jax.jit
def kernel(x, edge_index, W1, b1, W2, b2):
  n = x.shape[0]
  e = edge_index.shape[1]

  nch = -(-e // (_NW * _CH))
  nch = ((nch + 7) // 8) * 8
  e_pad = _NW * _CH * nch
  n_pad = ((n + 1 + _NS * 8 - 1) // (_NS * 8)) * (_NS * 8)
  src = edge_index[0].astype(jnp.int32)
  dst = edge_index[1].astype(jnp.int32)
  if e_pad != e:
    fill = jnp.full((e_pad - e,), n, dtype=jnp.int32)
    src = jnp.concatenate([src, fill])
    dst = jnp.concatenate([dst, fill])
  eidx = jnp.stack([src, dst]).reshape(2, _NW, nch, 1, _CH)
  zeros_nd = jnp.zeros((n_pad, 128), jnp.float32)
  ones_chd = jnp.ones((_CH, 128), jnp.float32)

  degp = _sc_degree(eidx, zeros_nd, ones_chd, n_pad, nch)
  hs1 = _tc_first(x, W1, degp, n_pad)
  aggp1 = _sc_aggregate(hs1, eidx, n_pad, nch)
  hs2 = _tc_mid(aggp1, hs1, degp, b1.reshape(1, -1), W2, n)
  aggp2 = _sc_aggregate(hs2, eidx, n_pad, nch)
  return _tc_last(aggp2, hs2, degp, b2.reshape(1, -1), n)


# speedup vs baseline: 1.3961x; 1.0693x over previous
import functools

import jax
import jax.numpy as jnp
from jax import lax
from jax.experimental import pallas as pl
from jax.experimental.pallas import tpu as pltpu
from jax.experimental.pallas import tpu_sc as plsc

_NC = 2
_NS = 16
_NW = _NC * _NS
_CH = 64


def _sc_mesh():
  return plsc.VectorSubcoreMesh(core_axis_name="c", subcore_axis_name="s")


_NB = 2


def _sc_aggregate(hs, eidx, n_pad, nch):
  d = hs.shape[1]
  rpt = n_pad // _NS
  half = nch // 2

  @functools.partial(
      pl.kernel,
      out_type=jax.ShapeDtypeStruct((_NC, n_pad, d), jnp.float32),
      mesh=_sc_mesh(),
      scratch_types=[
          pltpu.VMEM_SHARED((n_pad, d), jnp.float32),
          pltpu.VMEM((2, half, _CH), jnp.int32),
          pltpu.VMEM((_NB, _CH, d), jnp.float32),
          pltpu.SemaphoreType.DMA,
          pltpu.SemaphoreType.DMA,
      ],
  )
  def k(hs_hbm, eidx_hbm, out_hbm, agg_sh, idx, bufs, s0, s1):
    sems = (s0, s1)
    c = lax.axis_index("c")
    s = lax.axis_index("s")
    w = c * _NS + s
    pltpu.sync_copy(hs_hbm.at[pl.ds(s * rpt, rpt)],
                    agg_sh.at[pl.ds(s * rpt, rpt)])
    plsc.subcore_barrier()

    for p in range(2):
      pltpu.sync_copy(eidx_hbm.at[w, :, pl.ds(p * half, half)], idx)
      for b in range(_NB):
        pltpu.async_copy(hs_hbm.at[idx.at[0, b]], bufs.at[b], sems[b])

      @pl.loop(0, half, step=_NB)
      def _(j):
        for b in range(_NB):
          pltpu.make_async_copy(hs_hbm.at[idx.at[0, j + b]], bufs.at[b],
                                sems[b]).wait()
          pltpu.sync_copy(bufs.at[b], agg_sh.at[idx.at[1, j + b]], add=True)

          @pl.when(j + b + _NB < half)
          def _():
            pltpu.async_copy(hs_hbm.at[idx.at[0, j + b + _NB]], bufs.at[b],
                             sems[b])

    plsc.subcore_barrier()
    pltpu.sync_copy(agg_sh.at[pl.ds(s * rpt, rpt)],
                    out_hbm.at[c, pl.ds(s * rpt, rpt)])

  return k(hs, eidx)


def _sc_degree(eidx, zeros_nd, ones_chd, n_pad, nch):
  dd = zeros_nd.shape[1]
  rpt = n_pad // _NS

  @functools.partial(
      pl.kernel,
      out_type=jax.ShapeDtypeStruct((_NC, n_pad, dd), jnp.float32),
      mesh=_sc_mesh(),
      scratch_types=[
          pltpu.VMEM_SHARED((n_pad, dd), jnp.float32),
          pltpu.VMEM((nch, _CH), jnp.int32),
          pltpu.VMEM((_CH, dd), jnp.float32),
      ],
  )
  def k(eidx_hbm, zeros_hbm, ones_hbm, out_hbm, agg_sh, idx, ones_buf):
    c = lax.axis_index("c")
    s = lax.axis_index("s")
    w = c * _NS + s
    pltpu.sync_copy(eidx_hbm.at[w, 1], idx)
    pltpu.sync_copy(zeros_hbm.at[pl.ds(s * rpt, rpt)],
                    agg_sh.at[pl.ds(s * rpt, rpt)])
    pltpu.sync_copy(ones_hbm, ones_buf)
    plsc.subcore_barrier()

    @pl.loop(0, nch)
    def _(j):
      pltpu.sync_copy(ones_buf, agg_sh.at[idx.at[j]], add=True)

    plsc.subcore_barrier()
    pltpu.sync_copy(agg_sh.at[pl.ds(s * rpt, rpt)],
                    out_hbm.at[c, pl.ds(s * rpt, rpt)])

  return k(eidx, zeros_nd, ones_chd)


def _dinv_col(degp_a, degp_b, n):
  deg = degp_a + degp_b + 1.0
  return lax.rsqrt(jnp.maximum(deg, 1.0))[:n, 0:1]


def _tc_first(x, w1, degp, n_pad):
  n, d_in = x.shape
  d = w1.shape[1]

  def body(x_ref, w_ref, degp_ref, hs_ref):
    dinv = _dinv_col(degp_ref[0], degp_ref[1], n)
    h = jnp.dot(x_ref[...], w_ref[...], preferred_element_type=jnp.float32)
    hs_ref[0:n, :] = h * dinv
    hs_ref[n:n_pad, :] = jnp.zeros((n_pad - n, d), jnp.float32)

  return pl.pallas_call(
      body,
      out_shape=jax.ShapeDtypeStruct((n_pad, d), jnp.float32),
  )(x, w1, degp)


def _tc_mid(aggp, hs1, degp, b1, w2, n):
  n_pad, d = hs1.shape

  def body(aggp_ref, hs1_ref, degp_ref, b_ref, w_ref, hs2_ref):
    dinv_full = lax.rsqrt(
        jnp.maximum(degp_ref[0] + degp_ref[1] + 1.0, 1.0))[:, 0:1]
    agg = aggp_ref[0] + aggp_ref[1] - hs1_ref[...]
    h1 = jnp.maximum(dinv_full * agg + b_ref[...], 0.0)
    hs2 = jnp.dot(h1, w_ref[...],
                  preferred_element_type=jnp.float32) * dinv_full
    hs2_ref[0:n, :] = hs2[0:n, :]
    hs2_ref[n:n_pad, :] = jnp.zeros((n_pad - n, d), jnp.float32)

  return pl.pallas_call(
      body,
      out_shape=jax.ShapeDtypeStruct((n_pad, d), jnp.float32),
  )(aggp, hs1, degp, b1, w2)


def _tc_last(aggp, hs2, degp, b2, n):
  n_pad, d = hs2.shape

  def body(aggp_ref, hs2_ref, degp_ref, b_ref, out_ref):
    dinv = _dinv_col(degp_ref[0], degp_ref[1], n)
    agg = (aggp_ref[0] + aggp_ref[1] - hs2_ref[...])[0:n, :]
    v = dinv * agg + b_ref[...]
    shifted = v - jnp.max(v, axis=1, keepdims=True)
    out_ref[...] = shifted - jnp.log(
        jnp.sum(jnp.exp(shifted), axis=1, keepdims=True))

  return pl.pallas_call(
      body,
      out_shape=jax.ShapeDtypeStruct((n, d), jnp.float32),
  )(aggp, hs2, degp, b2)


@jax.jit
def kernel(x, edge_index, W1, b1, W2, b2):
  n = x.shape[0]
  e = edge_index.shape[1]

  nch = -(-e // (_NW * _CH))
  nch = ((nch + 7) // 8) * 8
  e_pad = _NW * _CH * nch
  n_pad = ((n + 1 + _NS * 8 - 1) // (_NS * 8)) * (_NS * 8)
  src = edge_index[0].astype(jnp.int32)
  dst = edge_index[1].astype(jnp.int32)
  if e_pad != e:
    fill = jnp.full((e_pad - e,), n, dtype=jnp.int32)
    src = jnp.concatenate([src, fill])
    dst = jnp.concatenate([dst, fill])
  eidx = jnp.stack(
      [src.reshape(_NW, nch, _CH), dst.reshape(_NW, nch, _CH)], axis=1)
  zeros_nd = jnp.zeros((n_pad, 128), jnp.float32)
  ones_chd = jnp.ones((_CH, 128), jnp.float32)

  degp = _sc_degree(eidx, zeros_nd, ones_chd, n_pad, nch)
  hs1 = _tc_first(x, W1, degp, n_pad)
  aggp1 = _sc_aggregate(hs1, eidx, n_pad, nch)
  hs2 = _tc_mid(aggp1, hs1, degp, b1.reshape(1, -1), W2, n)
  aggp2 = _sc_aggregate(hs2, eidx, n_pad, nch)
  return _tc_last(aggp2, hs2, degp, b2.reshape(1, -1), n)
